# baseline (speedup 1.0000x reference)
import functools

import jax
import jax.numpy as jnp
from jax import lax
from jax.experimental import pallas as pl
from jax.experimental.pallas import tpu as pltpu
from jax.experimental.pallas import tpu_sc as plsc

N = 6144
M = 8192
NK = 32
F32 = jnp.float32
BF16 = jnp.bfloat16
I32 = jnp.int32


def _mmd(a, b):
    return jax.lax.dot_general(a.astype(BF16), b.astype(BF16),
                               (((a.ndim - 1,), (0,)), ((), ())),
                               preferred_element_type=F32)


def _relu(x):
    return jnp.maximum(x, 0.0)


def _stn_feat_kernel(x_ref, w1, b1, w2, b2, w3, b3, g_ref):
    h = _relu(_mmd(x_ref[...], w1[...]) + b1[...])
    h = _relu(_mmd(h, w2[...]) + b2[...])
    h = _relu(_mmd(h, w3[...]) + b3[...])
    m = jnp.max(h, axis=0, keepdims=True)

    @pl.when(pl.program_id(0) == 0)
    def _():
        g_ref[...] = m

    @pl.when(pl.program_id(0) > 0)
    def _():
        g_ref[...] = jnp.maximum(g_ref[...], m)


def _stn_feat(x, p):
    R = 1024
    full = lambda s: pl.BlockSpec(s, lambda i: (0, 0))
    return pl.pallas_call(
        _stn_feat_kernel,
        grid=(N // R,),
        in_specs=[pl.BlockSpec((R, 24), lambda i: (i, 0)),
                  full((24, 64)), full((1, 64)),
                  full((64, 128)), full((1, 128)),
                  full((128, 1024)), full((1, 1024))],
        out_specs=pl.BlockSpec((1, 1024), lambda i: (0, 0)),
        out_shape=jax.ShapeDtypeStruct((1, 1024), F32),
    )(x, p["stn_c1"]["W"], p["stn_c1"]["b"][None, :],
      p["stn_c2"]["W"], p["stn_c2"]["b"][None, :],
      p["stn_c3"]["W"], p["stn_c3"]["b"][None, :])


def _stn_head_kernel(g_ref, w1, b1, w2, b2, w3, b3, eye_ref, t_ref):
    h = _relu(_mmd(g_ref[...], w1[...]) + b1[...])
    h = _relu(_mmd(h, w2[...]) + b2[...])
    t_ref[...] = _mmd(h, w3[...]) + b3[...] + eye_ref[...]


def _stn_head(g, p):
    eye_flat = jnp.eye(24, dtype=F32).reshape(1, 576)
    full = lambda s: pl.BlockSpec(s, lambda: (0, 0))
    t = pl.pallas_call(
        _stn_head_kernel,
        in_specs=[full((1, 1024)),
                  full((1024, 512)), full((1, 512)),
                  full((512, 256)), full((1, 256)),
                  full((256, 576)), full((1, 576)), full((1, 576))],
        out_specs=full((1, 576)),
        out_shape=jax.ShapeDtypeStruct((1, 576), F32),
    )(g, p["stn_f1"]["W"], p["stn_f1"]["b"][None, :],
      p["stn_f2"]["W"], p["stn_f2"]["b"][None, :],
      p["stn_f3"]["W"], p["stn_f3"]["b"][None, :], eye_flat)
    return t.reshape(24, 24)


def _xt_p1_kernel(x_ref, t_ref, w1a, b1, xt_ref, p_ref):
    xt = _mmd(x_ref[...], t_ref[...])
    R = xt.shape[0]
    xt_ref[...] = jnp.concatenate([xt, jnp.zeros((R, 104), F32)], axis=1)
    p_ref[...] = _mmd(xt, w1a[...]) + b1[...]


def _xt_p1(x, T, w1a, b1):
    R = 512
    full = lambda s: pl.BlockSpec(s, lambda i: (0, 0))
    return pl.pallas_call(
        _xt_p1_kernel,
        grid=(N // R,),
        in_specs=[pl.BlockSpec((R, 24), lambda i: (i, 0)),
                  full((24, 24)), full((24, 24)), full((1, 24))],
        out_specs=[pl.BlockSpec((R, 128), lambda i: (i, 0)),
                   pl.BlockSpec((R, 24), lambda i: (i, 0))],
        out_shape=[jax.ShapeDtypeStruct((N, 128), F32),
                   jax.ShapeDtypeStruct((N, 24), F32)],
    )(x, T, w1a, b1)


def _bitonic_argsort(d, ii):
    nbits = d.shape[-1].bit_length() - 1

    def one_pass(d, idx, k, j):
        dd = jnp.left_shift(jnp.int32(1), j)
        is_left = (ii & dd) == 0
        flip = ((jax.lax.shift_right_logical(ii, k + 1)
                 ^ jax.lax.shift_right_logical(ii, j)) & 1) == 1
        pv = jnp.where(is_left, pltpu.roll(d, M - dd, 1), pltpu.roll(d, dd, 1))
        pi = jnp.where(is_left, pltpu.roll(idx, M - dd, 1),
                       pltpu.roll(idx, dd, 1))
        pb = (pv < d) | ((pv == d) & (pi < idx))
        take = pb ^ flip
        return jnp.where(take, pv, d), jnp.where(take, pi, idx)

    def outer(k, carry):
        def inner(i, carry):
            d, idx = carry
            return one_pass(d, idx, k, k - i)
        return lax.fori_loop(0, k + 1, inner, carry)

    d, idx = lax.fori_loop(0, nbits, outer, (d, ii))
    return d, idx


def _sort_kernel(prow_ref, post_ref, out_ref):
    pr = prow_ref[...]
    pt = post_ref[...]
    rn = pr[:, 3:4]
    cn = pt[3:4, :]
    d = rn + cn - 2.0 * _mmd(pr[:, :3], pt[:3, :])
    R = d.shape[0]
    d = jnp.concatenate(
        [d, jnp.full((R, M - N), jnp.inf, F32)], axis=1)
    _, idx = _bitonic_argsort(d, jax.lax.broadcasted_iota(I32, (R, M), 1))
    out_ref[...] = idx[:, :2432]


_SORT_R = 16


def _sorted_neighbors(pos_a, pos_at):
    R = _SORT_R
    return pl.pallas_call(
        _sort_kernel,
        grid=(N // R,),
        in_specs=[pl.BlockSpec((R, 8), lambda i: (i, 0)),
                  pl.BlockSpec((8, N), lambda i: (0, 0))],
        out_specs=pl.BlockSpec((R, 2432), lambda i: (i, 0)),
        out_shape=jax.ShapeDtypeStruct((N, 2432), I32),
    )(pos_a, pos_at)


def _knn_kernel(xb_ref, xt_ref, out_ref):
    xb = xb_ref[...]
    xt = xt_ref[...]
    rn = xb[:, 24:25]
    cn = xt[24:25, :]
    d0 = rn + cn - 2.0 * _mmd(xb[:, :24], xt[:24, :])
    R = d0.shape[0]
    ii = jax.lax.broadcasted_iota(I32, (R, M), 1)
    ii32 = jax.lax.broadcasted_iota(I32, (R, 32), 1)

    def step(t, carry):
        d, out = carry
        mn = jnp.min(d, axis=1, keepdims=True)
        cand = jnp.where(d == mn, ii, M)
        a = jnp.min(cand, axis=1, keepdims=True)
        out = jnp.where(ii32 == (t - 1), a, out)
        return jnp.where(ii == a, jnp.inf, d), out

    d = jnp.concatenate([d0, jnp.full((R, M - N), jnp.inf, F32)], axis=1)
    _, out = lax.fori_loop(0, 33, step, (d, jnp.zeros((R, 32), I32)))
    out_ref[...] = out


def _knn33(x_a, x_at):
    R = 32
    return pl.pallas_call(
        _knn_kernel,
        grid=(N // R,),
        in_specs=[pl.BlockSpec((R, 32), lambda i: (i, 0)),
                  pl.BlockSpec((32, N), lambda i: (0, 0))],
        out_specs=pl.BlockSpec((R, 32), lambda i: (i, 0)),
        out_shape=jax.ShapeDtypeStruct((N, 32), I32),
    )(x_a, x_at)


def _make_sc_gather(n_idx, d, dtype):
    NW = 32
    per = n_idx // NW
    C = 128
    chunks = per // C
    mesh = plsc.VectorSubcoreMesh(core_axis_name="c", subcore_axis_name="s")

    @functools.partial(
        pl.kernel,
        out_type=jax.ShapeDtypeStruct((n_idx, d), dtype),
        mesh=mesh,
        scratch_types=[pltpu.VMEM((C,), I32),
                       pltpu.VMEM((C, d), dtype),
                       pltpu.SemaphoreType.DMA],
    )
    def gather(table_hbm, idx_hbm, out_hbm, idx_v, rows_v, sem):
        wid = lax.axis_index("s") * 2 + lax.axis_index("c")
        base = wid * per

        def body(i, carry):
            off = base + i * C
            pltpu.sync_copy(idx_hbm.at[pl.ds(off, C)], idx_v)
            pltpu.async_copy(table_hbm.at[idx_v], rows_v, sem).wait()
            pltpu.sync_copy(rows_v, out_hbm.at[pl.ds(off, C)])
            return carry

        lax.fori_loop(0, chunks, body, 0)

    return gather


@functools.cache
def _sc_gather_cached(n_idx, d, dtype):
    return _make_sc_gather(n_idx, d, dtype)


def _gather_rows(table, idx_flat, kind):
    if kind == "conv":
        return _sc_gather_cached(N * NK, 128, F32)(table, idx_flat)
    return _sc_gather_cached(N * 12, 128, F32)(table, idx_flat)


def _edge_h(p, g, xi, w1b, do):
    R = p.shape[0]
    dif = g[..., :do] - xi[:, None, :]
    q = _mmd(dif.reshape(R * NK, do), w1b)
    q = q.reshape(R, NK, q.shape[-1])
    return _relu(p[:, None, :] + q)


def _comb_kernel(do, has_next, a_ref, g_ref, xi_ref, w1b, w2, b2, *rest):
    h = _edge_h(a_ref[...], g_ref[...], xi_ref[...], w1b[...], do)
    hm = jnp.max(h, axis=1)
    x = _relu(_mmd(hm, w2[...]) + b2[...])
    R = x.shape[0]
    if has_next:
        w1a_n, b1_n, x_ref, x128_ref, pn_ref = rest
        x_ref[...] = x
        x128_ref[...] = jnp.concatenate(
            [x, jnp.zeros((R, 128 - x.shape[1]), F32)], axis=1)
        pn_ref[...] = _mmd(x, w1a_n[...]) + b1_n[...]
    else:
        rest[0][...] = x


def _comb_egc(a, g, xi, w1b, w2, b2, extra):
    R = 256
    rspec = lambda c: pl.BlockSpec((R, c), lambda i: (i, 0))
    full = lambda s: pl.BlockSpec(s, lambda i: (0, 0))
    in_specs = [rspec(24), pl.BlockSpec((R, NK, 128), lambda i: (i, 0, 0)),
                rspec(24), full((24, 24)), full((24, 24)), full((1, 24))]
    args = [a, g, xi, w1b, w2, b2]
    w1a_n, b1_n = extra
    in_specs += [full((24, 24)), full((1, 24))]
    args += [w1a_n, b1_n]
    out_specs = [rspec(24), rspec(128), rspec(24)]
    out_shape = [jax.ShapeDtypeStruct((N, 24), F32),
                 jax.ShapeDtypeStruct((N, 128), F32),
                 jax.ShapeDtypeStruct((N, 24), F32)]
    return pl.pallas_call(
        functools.partial(_comb_kernel, 24, True),
        grid=(N // R,),
        in_specs=in_specs, out_specs=out_specs, out_shape=out_shape,
    )(*args)


def _comb_dil_kernel(has_next, a_ref, g_ref, xi_ref, w1b, w2, b2, *rest):
    h = _edge_h(a_ref[...], g_ref[...], xi_ref[...], w1b[...], 60)
    hm = jnp.max(h, axis=1)
    x = _relu(_mmd(hm, w2[...]) + b2[...]) + xi_ref[...]
    R = x.shape[0]
    if has_next:
        w1a_n, b1_n, x_ref, x128_ref, pn_ref = rest
        x_ref[...] = x
        x128_ref[...] = jnp.concatenate(
            [x, jnp.zeros((R, 68), F32)], axis=1)
        pn_ref[...] = _mmd(x, w1a_n[...]) + b1_n[...]
    else:
        rest[0][...] = x


def _comb_dilated(a, g, xi, w1b, w2, b2, extra):
    R = 256
    rspec = lambda c: pl.BlockSpec((R, c), lambda i: (i, 0))
    full = lambda s: pl.BlockSpec(s, lambda i: (0, 0))
    in_specs = [rspec(60), pl.BlockSpec((R, NK, 128), lambda i: (i, 0, 0)),
                rspec(60), full((60, 60)), full((60, 60)), full((1, 60))]
    args = [a, g, xi, w1b, w2, b2]
    if extra is not None:
        w1a_n, b1_n = extra
        in_specs += [full((60, 60)), full((1, 60))]
        args += [w1a_n, b1_n]
        out_specs = [rspec(60), rspec(128), rspec(60)]
        out_shape = [jax.ShapeDtypeStruct((N, 60), F32),
                     jax.ShapeDtypeStruct((N, 128), F32),
                     jax.ShapeDtypeStruct((N, 60), F32)]
        has_next = True
    else:
        out_specs = [rspec(60)]
        out_shape = [jax.ShapeDtypeStruct((N, 60), F32)]
        has_next = False
    return pl.pallas_call(
        functools.partial(_comb_dil_kernel, has_next),
        grid=(N // R,),
        in_specs=in_specs, out_specs=out_specs, out_shape=out_shape,
    )(*args)


def _comb3_kernel(a_ref, g_ref, xi_ref, w1b, w2, b2, x1_ref, wlh, blh,
                  wd1a, bd1, xloc_ref, xmid_ref, pd_ref, xm128_ref):
    h = _edge_h(a_ref[...], g_ref[...], xi_ref[...], w1b[...], 24)
    hm = jnp.max(h, axis=1)
    x3 = _relu(_mmd(hm, w2[...]) + b2[...])
    xloc = jnp.concatenate([x1_ref[...], xi_ref[...], x3], axis=1)
    xmid = _relu(_mmd(xloc, wlh[...]) + blh[...])
    R = xmid.shape[0]
    xloc_ref[...] = xloc
    xmid_ref[...] = xmid
    pd_ref[...] = _mmd(xmid, wd1a[...]) + bd1[...]
    xm128_ref[...] = jnp.concatenate(
        [xmid, jnp.zeros((R, 68), F32)], axis=1)


def _comb3(a, g, x2, w1b, w2, b2, x1, wlh, blh, wd1a, bd1):
    R = 256
    rspec = lambda c: pl.BlockSpec((R, c), lambda i: (i, 0))
    full = lambda s: pl.BlockSpec(s, lambda i: (0, 0))
    return pl.pallas_call(
        _comb3_kernel,
        grid=(N // R,),
        in_specs=[rspec(24), pl.BlockSpec((R, NK, 128), lambda i: (i, 0, 0)),
                  rspec(24), full((24, 24)), full((24, 24)), full((1, 24)),
                  rspec(24), full((72, 60)), full((1, 60)),
                  full((60, 60)), full((1, 60))],
        out_specs=[rspec(72), rspec(60), rspec(60), rspec(128)],
        out_shape=[jax.ShapeDtypeStruct((N, 72), F32),
                   jax.ShapeDtypeStruct((N, 60), F32),
                   jax.ShapeDtypeStruct((N, 60), F32),
                   jax.ShapeDtypeStruct((N, 128), F32)],
    )(a, g, x2, w1b, w2, b2, x1, wlh, blh, wd1a, bd1)


def _head_a_kernel(xmid, xd1, xd2, xd3, xd4, w1, b1, lng, lnb, w2, b2,
                   logits_ref, tgt_ref):
    xt = jnp.concatenate([xmid[...], xd1[...], xd2[...], xd3[...], xd4[...]],
                         axis=1)
    t = _mmd(xt, w1[...]) + b1[...]
    mu = jnp.mean(t, axis=1, keepdims=True)
    v = jnp.mean((t - mu) ** 2, axis=1, keepdims=True)
    t = (t - mu) / jnp.sqrt(v + 1e-5) * lng[...] + lnb[...]
    logits = _mmd(_relu(t), w2[...]) + b2[...]
    logits_ref[...] = logits
    mx = jnp.max(logits, axis=1, keepdims=True)
    i17 = jax.lax.broadcasted_iota(I32, logits.shape, 1)
    tgt = jnp.min(jnp.where(logits == mx, i17, 17), axis=1, keepdims=True)
    tgt_ref[...] = jnp.broadcast_to(tgt.astype(F32), tgt_ref.shape)


def _head_a(xmid, xd1, xd2, xd3, xd4, p):
    R = 256
    rspec = lambda c: pl.BlockSpec((R, c), lambda i: (i, 0))
    full = lambda s: pl.BlockSpec(s, lambda i: (0, 0))
    return pl.pallas_call(
        _head_a_kernel,
        grid=(N // R,),
        in_specs=[rspec(60)] * 5 + [full((300, 160)), full((1, 160)),
                                    full((1, 160)), full((1, 160)),
                                    full((160, 17)), full((1, 17))],
        out_specs=[rspec(17), rspec(8)],
        out_shape=[jax.ShapeDtypeStruct((N, 17), F32),
                   jax.ShapeDtypeStruct((N, 8), F32)],
    )(xmid, xd1, xd2, xd3, xd4, p["tc1"]["W"], p["tc1"]["b"][None, :],
      p["tc_ln_g"][None, :], p["tc_ln_b"][None, :],
      p["tc2"]["W"], p["tc2"]["b"][None, :])


def _head_c_kernel(xloc, xmid, xd1, xd2, xd3, xd4, logits_ref, tgt_ref,
                   nb_ref, pos_ref,
                   fp0w, fp0b, fp1w, fp1b, fp2w, fp2b,
                   be1w, be1b, be2w, be2b, at1w, at1b, at2w, at2b,
                   op1w, op1b, op2w, op2b, fiw, fib,
                   r11w, r11b, r12w, r12b, r1rw, r1rb,
                   r21w, r21b, r22w, r22b, r2rw, r2rb,
                   outw, outb, out_ref):
    logits = logits_ref[...]
    tgt = tgt_ref[..., 0:1]
    nlab = nb_ref[..., 3]
    diff = (nlab != tgt).astype(F32)
    dr = jnp.mean(diff, axis=1, keepdims=True)
    dx = nb_ref[..., 0] - pos_ref[..., 0:1]
    dy = nb_ref[..., 1] - pos_ref[..., 1:2]
    dz = nb_ref[..., 2] - pos_ref[..., 2:3]
    dist = jnp.sqrt(dx * dx + dy * dy + dz * dz)
    same = 1.0 - diff
    same_dist = jnp.sum(dist * same, axis=1, keepdims=True) / (
        jnp.sum(same, axis=1, keepdims=True) + 1e-6)
    bdist = jnp.min(jnp.where(diff > 0.0, dist, jnp.inf), axis=1,
                    keepdims=True)
    bdist = jnp.where(jnp.isfinite(bdist), bdist, same_dist)
    dmean = jnp.mean(dist, axis=1, keepdims=True)
    density = 1.0 / (dmean + 1e-6)
    var1 = jnp.sum((dist - dmean) ** 2, axis=1, keepdims=True) / 11.0
    curv = jnp.sqrt(var1) / (dmean + 1e-6)
    s = logits / 0.75
    s = s - jnp.max(s, axis=1, keepdims=True)
    es = jnp.exp(s)
    probs = es / jnp.sum(es, axis=1, keepdims=True)
    conf = jnp.max(probs, axis=1, keepdims=True)
    ent = -jnp.sum(probs * jnp.log(probs + 1e-8), axis=1, keepdims=True) / \
        jnp.log(jnp.float32(17.0))
    binfo = jnp.concatenate([dr, conf, ent, density, curv, bdist], axis=1)
    benc = _relu(_mmd(_relu(_mmd(binfo, be1w[...]) + be1b[...]), be2w[...])
                 + be2b[...])
    xglob = jnp.concatenate([xd1[...], xd2[...], xd3[...], xd4[...]], axis=1)
    f0 = _mmd(xloc[...], fp0w[...]) + fp0b[...]
    f1 = _mmd(xmid[...], fp1w[...]) + fp1b[...]
    f2 = _mmd(xglob, fp2w[...]) + fp2b[...]
    gf = (f0 + f1 + f2) / 3.0
    ah = _relu(_mmd(jnp.concatenate([gf, benc], axis=1), at1w[...])
               + at1b[...])
    al = _mmd(ah, at2w[...]) + at2b[...]
    al = al - jnp.max(al, axis=1, keepdims=True)
    ae = jnp.exp(al)
    aw = ae / jnp.sum(ae, axis=1, keepdims=True)
    fused = f0 * aw[:, 0:1] + f1 * aw[:, 1:2] + f2 * aw[:, 2:3]
    x_fused = _mmd(_relu(_mmd(fused, op1w[...]) + op1b[...]), op2w[...]) \
        + op2b[...] + gf
    gate = _mmd(x_fused, fiw[...]) + fib[...]
    xf = x_fused * (1.0 / (1.0 + jnp.exp(-gate)))
    h1 = _relu(_mmd(xf, r11w[...]) + r11b[...])
    xf = _relu(_mmd(h1, r12w[...]) + r12b[...]
               + _mmd(xf, r1rw[...]) + r1rb[...])
    h2 = _relu(_mmd(xf, r21w[...]) + r21b[...])
    xf = _relu(_mmd(h2, r22w[...]) + r22b[...]
               + _mmd(xf, r2rw[...]) + r2rb[...])
    out_ref[...] = _mmd(xf, outw[...]) + outb[...]


def _head_c(xloc, xmid, xd, logits, tgtf, nb_g, pos_pad, p):
    R = 256
    rspec = lambda c: pl.BlockSpec((R, c), lambda i: (i, 0))
    r3 = lambda c: pl.BlockSpec((R, 12, c), lambda i: (i, 0, 0))
    full = lambda s: pl.BlockSpec(s, lambda i: (0, 0))

    def wb(name):
        w = p[name]["W"]
        return [(w, full(w.shape)), (p[name]["b"][None, :],
                                     full((1, w.shape[1])))]

    wargs = []
    for nm in ["fp0", "fp1", "fp2", "be1", "be2", "at1", "at2",
               "op1", "op2", "fi", "r1_1", "r1_2", "r1_r",
               "r2_1", "r2_2", "r2_r", "out"]:
        wargs += wb(nm)
    return pl.pallas_call(
        _head_c_kernel,
        grid=(N // R,),
        in_specs=[rspec(72), rspec(60), rspec(60), rspec(60), rspec(60),
                  rspec(60), rspec(17), rspec(8), r3(128),
                  rspec(16)] + [s for _, s in wargs],
        out_specs=rspec(17),
        out_shape=jax.ShapeDtypeStruct((N, 17), F32),
    )(xloc, xmid, xd[0], xd[1], xd[2], xd[3], logits, tgtf,
      nb_g, pos_pad, *[a for a, _ in wargs])


def _split_w(w, c):
    return w[:c], w[c:]


def kernel(x, pos, params):
    p = params
    x2d = x[0]
    pos2d = pos[0]
    pos_norm = jnp.sum(pos2d * pos2d, -1)
    pos_a = jnp.concatenate(
        [pos2d, pos_norm[:, None], jnp.zeros((N, 4), F32)], axis=1)
    pos_pad16 = jnp.concatenate([pos2d, jnp.zeros((N, 13), F32)], axis=1)

    g = _stn_feat(x2d, p)
    T = _stn_head(g, p)

    w1a_1, w1b_1 = _split_w(p["egc1_1"]["W"], 24)
    xt128, p1 = _xt_p1(x2d, T, w1a_1, p["egc1_1"]["b"][None, :])
    xt24 = xt128[:, :24]

    snn = _sorted_neighbors(pos_a, pos_a.T)
    idx_pos = snn[:, 1:33]
    nidx = snn[:, 1:13]
    idx_d1 = snn[:, 0:192:6]
    idx_d2 = snn[:, 0:896:28]
    idx_d3 = snn[:, 0:1792:56]
    idx_d4 = snn[:, 0:2400:75]

    def feat_aug(xf):
        an = jnp.sum(xf * xf, -1)
        return jnp.concatenate(
            [xf, an[:, None], jnp.zeros((N, 7), F32)], axis=1)

    g1 = _gather_rows(xt128, idx_pos.reshape(-1), "conv").reshape(N, NK, 128)
    w1a_2, w1b_2 = _split_w(p["egc2_1"]["W"], 24)
    x1, x1_128, p2 = _comb_egc(p1, g1, xt24, w1b_1, p["egc1_2"]["W"],
                               p["egc1_2"]["b"][None, :],
                               (w1a_2, p["egc2_1"]["b"][None, :]))
    x1a = feat_aug(x1)
    idx1 = _knn33(x1a, x1a.T)
    g2 = _gather_rows(x1_128, idx1.reshape(-1), "conv").reshape(N, NK, 128)
    w1a_3, w1b_3 = _split_w(p["egc3_1"]["W"], 24)
    x2, x2_128, p3 = _comb_egc(p2, g2, x1, w1b_2, p["egc2_2"]["W"],
                               p["egc2_2"]["b"][None, :],
                               (w1a_3, p["egc3_1"]["b"][None, :]))
    x2a = feat_aug(x2)
    idx2 = _knn33(x2a, x2a.T)
    g3 = _gather_rows(x2_128, idx2.reshape(-1), "conv").reshape(N, NK, 128)
    w1a_d1, w1b_d1 = _split_w(p["d1_1"]["W"], 60)
    xloc, xmid, p_d1, xmid128 = _comb3(
        p3, g3, x2, w1b_3, p["egc3_2"]["W"], p["egc3_2"]["b"][None, :], x1,
        p["local_hidden"]["W"], p["local_hidden"]["b"][None, :],
        w1a_d1, p["d1_1"]["b"][None, :])

    idx_all = [idx_d1, idx_d2, idx_d3, idx_d4]
    w1bs = {"d1": w1b_d1}
    for nm in ["d2", "d3", "d4"]:
        _, w1bs[nm] = _split_w(p[nm + "_1"]["W"], 60)
    feat = xmid
    table = xmid128
    p_cur = p_d1
    xds = []
    for i, nm in enumerate(["d1", "d2", "d3", "d4"]):
        gd = _gather_rows(table, idx_all[i].reshape(-1), "conv").reshape(
            N, NK, 128)
        if i < 3:
            nxt = ["d2", "d3", "d4"][i]
            w1a_n, _ = _split_w(p[nxt + "_1"]["W"], 60)
            xd, table, p_cur = _comb_dilated(
                p_cur, gd, feat, w1bs[nm], p[nm + "_2"]["W"],
                p[nm + "_2"]["b"][None, :],
                (w1a_n, p[nxt + "_1"]["b"][None, :]))
        else:
            (xd,) = _comb_dilated(
                p_cur, gd, feat, w1bs[nm], p[nm + "_2"]["W"],
                p[nm + "_2"]["b"][None, :], None)
        xds.append(xd)
        feat = xd

    logits, tgtf = _head_a(xmid, xds[0], xds[1], xds[2], xds[3], p)
    nb_table = jnp.concatenate(
        [pos2d, tgtf[:, 0:1], jnp.zeros((N, 124), F32)], axis=1)
    nb_g = _gather_rows(nb_table, nidx.reshape(-1), "small").reshape(
        N, 12, 128)

    out = _head_c(xloc, xmid, xds, logits, tgtf, nb_g, pos_pad16, p)
    return out[None]


# speedup vs baseline: 1.3268x; 1.3268x over previous
import functools

import jax
import jax.numpy as jnp
from jax import lax
from jax.experimental import pallas as pl
from jax.experimental.pallas import tpu as pltpu
from jax.experimental.pallas import tpu_sc as plsc

N = 6144
M = 8192
NK = 32
F32 = jnp.float32
BF16 = jnp.bfloat16
I32 = jnp.int32


def _mmd(a, b):
    return jax.lax.dot_general(a.astype(BF16), b.astype(BF16),
                               (((a.ndim - 1,), (0,)), ((), ())),
                               preferred_element_type=F32)


def _relu(x):
    return jnp.maximum(x, 0.0)


def _stn_feat_kernel(x_ref, w1, b1, w2, b2, w3, b3, g_ref):
    h = _relu(_mmd(x_ref[...], w1[...]) + b1[...])
    h = _relu(_mmd(h, w2[...]) + b2[...])
    h = _relu(_mmd(h, w3[...]) + b3[...])
    m = jnp.max(h, axis=0, keepdims=True)

    @pl.when(pl.program_id(0) == 0)
    def _():
        g_ref[...] = m

    @pl.when(pl.program_id(0) > 0)
    def _():
        g_ref[...] = jnp.maximum(g_ref[...], m)


def _stn_feat(x, p):
    R = 1024
    full = lambda s: pl.BlockSpec(s, lambda i: (0, 0))
    return pl.pallas_call(
        _stn_feat_kernel,
        grid=(N // R,),
        in_specs=[pl.BlockSpec((R, 24), lambda i: (i, 0)),
                  full((24, 64)), full((1, 64)),
                  full((64, 128)), full((1, 128)),
                  full((128, 1024)), full((1, 1024))],
        out_specs=pl.BlockSpec((1, 1024), lambda i: (0, 0)),
        out_shape=jax.ShapeDtypeStruct((1, 1024), F32),
    )(x, p["stn_c1"]["W"], p["stn_c1"]["b"][None, :],
      p["stn_c2"]["W"], p["stn_c2"]["b"][None, :],
      p["stn_c3"]["W"], p["stn_c3"]["b"][None, :])


def _stn_head_kernel(g_ref, w1, b1, w2, b2, w3, b3, eye_ref, t_ref):
    h = _relu(_mmd(g_ref[...], w1[...]) + b1[...])
    h = _relu(_mmd(h, w2[...]) + b2[...])
    t_ref[...] = _mmd(h, w3[...]) + b3[...] + eye_ref[...]


def _stn_head(g, p):
    eye_flat = jnp.eye(24, dtype=F32).reshape(1, 576)
    full = lambda s: pl.BlockSpec(s, lambda: (0, 0))
    t = pl.pallas_call(
        _stn_head_kernel,
        in_specs=[full((1, 1024)),
                  full((1024, 512)), full((1, 512)),
                  full((512, 256)), full((1, 256)),
                  full((256, 576)), full((1, 576)), full((1, 576))],
        out_specs=full((1, 576)),
        out_shape=jax.ShapeDtypeStruct((1, 576), F32),
    )(g, p["stn_f1"]["W"], p["stn_f1"]["b"][None, :],
      p["stn_f2"]["W"], p["stn_f2"]["b"][None, :],
      p["stn_f3"]["W"], p["stn_f3"]["b"][None, :], eye_flat)
    return t.reshape(24, 24)


def _xt_p1_kernel(x_ref, t_ref, w1a, b1, xt_ref, p_ref):
    xt = _mmd(x_ref[...], t_ref[...])
    R = xt.shape[0]
    xt_ref[...] = jnp.concatenate([xt, jnp.zeros((R, 104), F32)], axis=1)
    p_ref[...] = _mmd(xt, w1a[...]) + b1[...]


def _xt_p1(x, T, w1a, b1):
    R = 512
    full = lambda s: pl.BlockSpec(s, lambda i: (0, 0))
    return pl.pallas_call(
        _xt_p1_kernel,
        grid=(N // R,),
        in_specs=[pl.BlockSpec((R, 24), lambda i: (i, 0)),
                  full((24, 24)), full((24, 24)), full((1, 24))],
        out_specs=[pl.BlockSpec((R, 128), lambda i: (i, 0)),
                   pl.BlockSpec((R, 24), lambda i: (i, 0))],
        out_shape=[jax.ShapeDtypeStruct((N, 128), F32),
                   jax.ShapeDtypeStruct((N, 24), F32)],
    )(x, T, w1a, b1)


_SORT_R = 16


def _cmpx(d, idx, pv, pi, flip):
    pb = (pv < d) | ((pv == d) & (pi < idx))
    take = pb ^ flip
    return jnp.where(take, pv, d), jnp.where(take, pi, idx)


def _pass_local(d, idx, ii_loc, kb, j):
    dd = 1 << j
    is_left = (ii_loc & dd) == 0
    flip = kb ^ ~is_left
    pv = jnp.where(is_left, jnp.roll(d, -dd, 1), jnp.roll(d, dd, 1))
    pi = jnp.where(is_left, jnp.roll(idx, -dd, 1), jnp.roll(idx, dd, 1))
    return _cmpx(d, idx, pv, pi, flip)


def _bitonic_argsort(d, ii):
    nbits = d.shape[-1].bit_length() - 1

    def stage(k, carry):
        d, idx = carry
        kb = (jax.lax.shift_right_logical(ii, k + 1) & 1) == 1
        for j in range(nbits - 1, -1, -1):
            d, idx = lax.cond(
                j <= k,
                lambda dx, ix, kk, jj=j: _pass_local(dx, ix, ii, kk, jj),
                lambda dx, ix, kk: (dx, ix),
                d, idx, kb)
        return d, idx

    return lax.fori_loop(0, nbits, stage, (d, ii))


def _sort_kernel(prow_ref, post_ref, out_ref):
    pr = prow_ref[...]
    pt = post_ref[...]
    rn = pr[:, 3:4]
    cn = pt[3:4, :]
    d = rn + cn - 2.0 * _mmd(pr[:, :3], pt[:3, :])
    R = d.shape[0]
    d = jnp.concatenate(
        [d, jnp.full((R, M - N), jnp.inf, F32)], axis=1)
    _, idx = _bitonic_argsort(d, jax.lax.broadcasted_iota(I32, (R, M), 1))
    out_ref[...] = idx[:, :2432]


def _sorted_neighbors(pos_a, pos_at):
    R = _SORT_R
    return pl.pallas_call(
        _sort_kernel,
        grid=(N // R,),
        in_specs=[pl.BlockSpec((R, 8), lambda i: (i, 0)),
                  pl.BlockSpec((8, N), lambda i: (0, 0))],
        out_specs=pl.BlockSpec((R, 2432), lambda i: (i, 0)),
        out_shape=jax.ShapeDtypeStruct((N, 2432), I32),
    )(pos_a, pos_at)


def _knn_kernel(xb_ref, xt_ref, out_ref):
    xb = xb_ref[...]
    xt = xt_ref[...]
    rn = xb[:, 24:25]
    cn = xt[24:25, :]
    d0 = rn + cn - 2.0 * _mmd(xb[:, :24], xt[:24, :])
    R = d0.shape[0]
    ii = jax.lax.broadcasted_iota(I32, (R, M), 1)
    ii32 = jax.lax.broadcasted_iota(I32, (R, 32), 1)

    def step(t, carry):
        d, out = carry
        mn = jnp.min(d, axis=1, keepdims=True)
        cand = jnp.where(d == mn, ii, M)
        a = jnp.min(cand, axis=1, keepdims=True)
        out = jnp.where(ii32 == (t - 1), a, out)
        return jnp.where(ii == a, jnp.inf, d), out

    d = jnp.concatenate([d0, jnp.full((R, M - N), jnp.inf, F32)], axis=1)
    _, out = lax.fori_loop(0, 33, step, (d, jnp.zeros((R, 32), I32)))
    out_ref[...] = out


def _knn33(x_a, x_at):
    R = 32
    return pl.pallas_call(
        _knn_kernel,
        grid=(N // R,),
        in_specs=[pl.BlockSpec((R, 32), lambda i: (i, 0)),
                  pl.BlockSpec((32, N), lambda i: (0, 0))],
        out_specs=pl.BlockSpec((R, 32), lambda i: (i, 0)),
        out_shape=jax.ShapeDtypeStruct((N, 32), I32),
    )(x_a, x_at)


def _make_sc_gather(n_idx, d, dtype):
    NW = 32
    per = n_idx // NW
    C = 128
    chunks = per // C
    mesh = plsc.VectorSubcoreMesh(core_axis_name="c", subcore_axis_name="s")

    @functools.partial(
        pl.kernel,
        out_type=jax.ShapeDtypeStruct((n_idx, d), dtype),
        mesh=mesh,
        scratch_types=[pltpu.VMEM((C,), I32),
                       pltpu.VMEM((C, d), dtype),
                       pltpu.SemaphoreType.DMA],
    )
    def gather(table_hbm, idx_hbm, out_hbm, idx_v, rows_v, sem):
        wid = lax.axis_index("s") * 2 + lax.axis_index("c")
        base = wid * per

        def body(i, carry):
            off = base + i * C
            pltpu.sync_copy(idx_hbm.at[pl.ds(off, C)], idx_v)
            pltpu.async_copy(table_hbm.at[idx_v], rows_v, sem).wait()
            pltpu.sync_copy(rows_v, out_hbm.at[pl.ds(off, C)])
            return carry

        lax.fori_loop(0, chunks, body, 0)

    return gather


@functools.cache
def _sc_gather_cached(n_idx, d, dtype):
    return _make_sc_gather(n_idx, d, dtype)


def _gather_rows(table, idx_flat, kind):
    if kind == "conv":
        return _sc_gather_cached(N * NK, 128, F32)(table, idx_flat)
    return _sc_gather_cached(N * 12, 128, F32)(table, idx_flat)


def _edge_h(p, g, xi, w1b, do):
    R = p.shape[0]
    dif = g[..., :do] - xi[:, None, :]
    q = _mmd(dif.reshape(R * NK, do), w1b)
    q = q.reshape(R, NK, q.shape[-1])
    return _relu(p[:, None, :] + q)


def _comb_kernel(do, has_next, a_ref, g_ref, xi_ref, w1b, w2, b2, *rest):
    h = _edge_h(a_ref[...], g_ref[...], xi_ref[...], w1b[...], do)
    hm = jnp.max(h, axis=1)
    x = _relu(_mmd(hm, w2[...]) + b2[...])
    R = x.shape[0]
    if has_next:
        w1a_n, b1_n, x_ref, x128_ref, pn_ref = rest
        x_ref[...] = x
        x128_ref[...] = jnp.concatenate(
            [x, jnp.zeros((R, 128 - x.shape[1]), F32)], axis=1)
        pn_ref[...] = _mmd(x, w1a_n[...]) + b1_n[...]
    else:
        rest[0][...] = x


def _comb_egc(a, g, xi, w1b, w2, b2, extra):
    R = 256
    rspec = lambda c: pl.BlockSpec((R, c), lambda i: (i, 0))
    full = lambda s: pl.BlockSpec(s, lambda i: (0, 0))
    in_specs = [rspec(24), pl.BlockSpec((R, NK, 128), lambda i: (i, 0, 0)),
                rspec(24), full((24, 24)), full((24, 24)), full((1, 24))]
    args = [a, g, xi, w1b, w2, b2]
    w1a_n, b1_n = extra
    in_specs += [full((24, 24)), full((1, 24))]
    args += [w1a_n, b1_n]
    out_specs = [rspec(24), rspec(128), rspec(24)]
    out_shape = [jax.ShapeDtypeStruct((N, 24), F32),
                 jax.ShapeDtypeStruct((N, 128), F32),
                 jax.ShapeDtypeStruct((N, 24), F32)]
    return pl.pallas_call(
        functools.partial(_comb_kernel, 24, True),
        grid=(N // R,),
        in_specs=in_specs, out_specs=out_specs, out_shape=out_shape,
    )(*args)


def _comb_dil_kernel(has_next, a_ref, g_ref, xi_ref, w1b, w2, b2, *rest):
    h = _edge_h(a_ref[...], g_ref[...], xi_ref[...], w1b[...], 60)
    hm = jnp.max(h, axis=1)
    x = _relu(_mmd(hm, w2[...]) + b2[...]) + xi_ref[...]
    R = x.shape[0]
    if has_next:
        w1a_n, b1_n, x_ref, x128_ref, pn_ref = rest
        x_ref[...] = x
        x128_ref[...] = jnp.concatenate(
            [x, jnp.zeros((R, 68), F32)], axis=1)
        pn_ref[...] = _mmd(x, w1a_n[...]) + b1_n[...]
    else:
        rest[0][...] = x


def _comb_dilated(a, g, xi, w1b, w2, b2, extra):
    R = 256
    rspec = lambda c: pl.BlockSpec((R, c), lambda i: (i, 0))
    full = lambda s: pl.BlockSpec(s, lambda i: (0, 0))
    in_specs = [rspec(60), pl.BlockSpec((R, NK, 128), lambda i: (i, 0, 0)),
                rspec(60), full((60, 60)), full((60, 60)), full((1, 60))]
    args = [a, g, xi, w1b, w2, b2]
    if extra is not None:
        w1a_n, b1_n = extra
        in_specs += [full((60, 60)), full((1, 60))]
        args += [w1a_n, b1_n]
        out_specs = [rspec(60), rspec(128), rspec(60)]
        out_shape = [jax.ShapeDtypeStruct((N, 60), F32),
                     jax.ShapeDtypeStruct((N, 128), F32),
                     jax.ShapeDtypeStruct((N, 60), F32)]
        has_next = True
    else:
        out_specs = [rspec(60)]
        out_shape = [jax.ShapeDtypeStruct((N, 60), F32)]
        has_next = False
    return pl.pallas_call(
        functools.partial(_comb_dil_kernel, has_next),
        grid=(N // R,),
        in_specs=in_specs, out_specs=out_specs, out_shape=out_shape,
    )(*args)


def _comb3_kernel(a_ref, g_ref, xi_ref, w1b, w2, b2, x1_ref, wlh, blh,
                  wd1a, bd1, xloc_ref, xmid_ref, pd_ref, xm128_ref):
    h = _edge_h(a_ref[...], g_ref[...], xi_ref[...], w1b[...], 24)
    hm = jnp.max(h, axis=1)
    x3 = _relu(_mmd(hm, w2[...]) + b2[...])
    xloc = jnp.concatenate([x1_ref[...], xi_ref[...], x3], axis=1)
    xmid = _relu(_mmd(xloc, wlh[...]) + blh[...])
    R = xmid.shape[0]
    xloc_ref[...] = xloc
    xmid_ref[...] = xmid
    pd_ref[...] = _mmd(xmid, wd1a[...]) + bd1[...]
    xm128_ref[...] = jnp.concatenate(
        [xmid, jnp.zeros((R, 68), F32)], axis=1)


def _comb3(a, g, x2, w1b, w2, b2, x1, wlh, blh, wd1a, bd1):
    R = 256
    rspec = lambda c: pl.BlockSpec((R, c), lambda i: (i, 0))
    full = lambda s: pl.BlockSpec(s, lambda i: (0, 0))
    return pl.pallas_call(
        _comb3_kernel,
        grid=(N // R,),
        in_specs=[rspec(24), pl.BlockSpec((R, NK, 128), lambda i: (i, 0, 0)),
                  rspec(24), full((24, 24)), full((24, 24)), full((1, 24)),
                  rspec(24), full((72, 60)), full((1, 60)),
                  full((60, 60)), full((1, 60))],
        out_specs=[rspec(72), rspec(60), rspec(60), rspec(128)],
        out_shape=[jax.ShapeDtypeStruct((N, 72), F32),
                   jax.ShapeDtypeStruct((N, 60), F32),
                   jax.ShapeDtypeStruct((N, 60), F32),
                   jax.ShapeDtypeStruct((N, 128), F32)],
    )(a, g, x2, w1b, w2, b2, x1, wlh, blh, wd1a, bd1)


def _head_a_kernel(xmid, xd1, xd2, xd3, xd4, w1, b1, lng, lnb, w2, b2,
                   logits_ref, tgt_ref):
    xt = jnp.concatenate([xmid[...], xd1[...], xd2[...], xd3[...], xd4[...]],
                         axis=1)
    t = _mmd(xt, w1[...]) + b1[...]
    mu = jnp.mean(t, axis=1, keepdims=True)
    v = jnp.mean((t - mu) ** 2, axis=1, keepdims=True)
    t = (t - mu) / jnp.sqrt(v + 1e-5) * lng[...] + lnb[...]
    logits = _mmd(_relu(t), w2[...]) + b2[...]
    logits_ref[...] = logits
    mx = jnp.max(logits, axis=1, keepdims=True)
    i17 = jax.lax.broadcasted_iota(I32, logits.shape, 1)
    tgt = jnp.min(jnp.where(logits == mx, i17, 17), axis=1, keepdims=True)
    tgt_ref[...] = jnp.broadcast_to(tgt.astype(F32), tgt_ref.shape)


def _head_a(xmid, xd1, xd2, xd3, xd4, p):
    R = 256
    rspec = lambda c: pl.BlockSpec((R, c), lambda i: (i, 0))
    full = lambda s: pl.BlockSpec(s, lambda i: (0, 0))
    return pl.pallas_call(
        _head_a_kernel,
        grid=(N // R,),
        in_specs=[rspec(60)] * 5 + [full((300, 160)), full((1, 160)),
                                    full((1, 160)), full((1, 160)),
                                    full((160, 17)), full((1, 17))],
        out_specs=[rspec(17), rspec(8)],
        out_shape=[jax.ShapeDtypeStruct((N, 17), F32),
                   jax.ShapeDtypeStruct((N, 8), F32)],
    )(xmid, xd1, xd2, xd3, xd4, p["tc1"]["W"], p["tc1"]["b"][None, :],
      p["tc_ln_g"][None, :], p["tc_ln_b"][None, :],
      p["tc2"]["W"], p["tc2"]["b"][None, :])


def _head_c_kernel(xloc, xmid, xd1, xd2, xd3, xd4, logits_ref, tgt_ref,
                   nb_ref, pos_ref,
                   fp0w, fp0b, fp1w, fp1b, fp2w, fp2b,
                   be1w, be1b, be2w, be2b, at1w, at1b, at2w, at2b,
                   op1w, op1b, op2w, op2b, fiw, fib,
                   r11w, r11b, r12w, r12b, r1rw, r1rb,
                   r21w, r21b, r22w, r22b, r2rw, r2rb,
                   outw, outb, out_ref):
    logits = logits_ref[...]
    tgt = tgt_ref[..., 0:1]
    nlab = nb_ref[..., 3]
    diff = (nlab != tgt).astype(F32)
    dr = jnp.mean(diff, axis=1, keepdims=True)
    dx = nb_ref[..., 0] - pos_ref[..., 0:1]
    dy = nb_ref[..., 1] - pos_ref[..., 1:2]
    dz = nb_ref[..., 2] - pos_ref[..., 2:3]
    dist = jnp.sqrt(dx * dx + dy * dy + dz * dz)
    same = 1.0 - diff
    same_dist = jnp.sum(dist * same, axis=1, keepdims=True) / (
        jnp.sum(same, axis=1, keepdims=True) + 1e-6)
    bdist = jnp.min(jnp.where(diff > 0.0, dist, jnp.inf), axis=1,
                    keepdims=True)
    bdist = jnp.where(jnp.isfinite(bdist), bdist, same_dist)
    dmean = jnp.mean(dist, axis=1, keepdims=True)
    density = 1.0 / (dmean + 1e-6)
    var1 = jnp.sum((dist - dmean) ** 2, axis=1, keepdims=True) / 11.0
    curv = jnp.sqrt(var1) / (dmean + 1e-6)
    s = logits / 0.75
    s = s - jnp.max(s, axis=1, keepdims=True)
    es = jnp.exp(s)
    probs = es / jnp.sum(es, axis=1, keepdims=True)
    conf = jnp.max(probs, axis=1, keepdims=True)
    ent = -jnp.sum(probs * jnp.log(probs + 1e-8), axis=1, keepdims=True) / \
        jnp.log(jnp.float32(17.0))
    binfo = jnp.concatenate([dr, conf, ent, density, curv, bdist], axis=1)
    benc = _relu(_mmd(_relu(_mmd(binfo, be1w[...]) + be1b[...]), be2w[...])
                 + be2b[...])
    xglob = jnp.concatenate([xd1[...], xd2[...], xd3[...], xd4[...]], axis=1)
    f0 = _mmd(xloc[...], fp0w[...]) + fp0b[...]
    f1 = _mmd(xmid[...], fp1w[...]) + fp1b[...]
    f2 = _mmd(xglob, fp2w[...]) + fp2b[...]
    gf = (f0 + f1 + f2) / 3.0
    ah = _relu(_mmd(jnp.concatenate([gf, benc], axis=1), at1w[...])
               + at1b[...])
    al = _mmd(ah, at2w[...]) + at2b[...]
    al = al - jnp.max(al, axis=1, keepdims=True)
    ae = jnp.exp(al)
    aw = ae / jnp.sum(ae, axis=1, keepdims=True)
    fused = f0 * aw[:, 0:1] + f1 * aw[:, 1:2] + f2 * aw[:, 2:3]
    x_fused = _mmd(_relu(_mmd(fused, op1w[...]) + op1b[...]), op2w[...]) \
        + op2b[...] + gf
    gate = _mmd(x_fused, fiw[...]) + fib[...]
    xf = x_fused * (1.0 / (1.0 + jnp.exp(-gate)))
    h1 = _relu(_mmd(xf, r11w[...]) + r11b[...])
    xf = _relu(_mmd(h1, r12w[...]) + r12b[...]
               + _mmd(xf, r1rw[...]) + r1rb[...])
    h2 = _relu(_mmd(xf, r21w[...]) + r21b[...])
    xf = _relu(_mmd(h2, r22w[...]) + r22b[...]
               + _mmd(xf, r2rw[...]) + r2rb[...])
    out_ref[...] = _mmd(xf, outw[...]) + outb[...]


def _head_c(xloc, xmid, xd, logits, tgtf, nb_g, pos_pad, p):
    R = 256
    rspec = lambda c: pl.BlockSpec((R, c), lambda i: (i, 0))
    r3 = lambda c: pl.BlockSpec((R, 12, c), lambda i: (i, 0, 0))
    full = lambda s: pl.BlockSpec(s, lambda i: (0, 0))

    def wb(name):
        w = p[name]["W"]
        return [(w, full(w.shape)), (p[name]["b"][None, :],
                                     full((1, w.shape[1])))]

    wargs = []
    for nm in ["fp0", "fp1", "fp2", "be1", "be2", "at1", "at2",
               "op1", "op2", "fi", "r1_1", "r1_2", "r1_r",
               "r2_1", "r2_2", "r2_r", "out"]:
        wargs += wb(nm)
    return pl.pallas_call(
        _head_c_kernel,
        grid=(N // R,),
        in_specs=[rspec(72), rspec(60), rspec(60), rspec(60), rspec(60),
                  rspec(60), rspec(17), rspec(8), r3(128),
                  rspec(16)] + [s for _, s in wargs],
        out_specs=rspec(17),
        out_shape=jax.ShapeDtypeStruct((N, 17), F32),
    )(xloc, xmid, xd[0], xd[1], xd[2], xd[3], logits, tgtf,
      nb_g, pos_pad, *[a for a, _ in wargs])


def _split_w(w, c):
    return w[:c], w[c:]


def kernel(x, pos, params):
    p = params
    x2d = x[0]
    pos2d = pos[0]
    pos_norm = jnp.sum(pos2d * pos2d, -1)
    pos_a = jnp.concatenate(
        [pos2d, pos_norm[:, None], jnp.zeros((N, 4), F32)], axis=1)
    pos_pad16 = jnp.concatenate([pos2d, jnp.zeros((N, 13), F32)], axis=1)

    g = _stn_feat(x2d, p)
    T = _stn_head(g, p)

    w1a_1, w1b_1 = _split_w(p["egc1_1"]["W"], 24)
    xt128, p1 = _xt_p1(x2d, T, w1a_1, p["egc1_1"]["b"][None, :])
    xt24 = xt128[:, :24]

    snn = _sorted_neighbors(pos_a, pos_a.T)
    idx_pos = snn[:, 1:33]
    nidx = snn[:, 1:13]
    idx_d1 = snn[:, 0:192:6]
    idx_d2 = snn[:, 0:896:28]
    idx_d3 = snn[:, 0:1792:56]
    idx_d4 = snn[:, 0:2400:75]

    def feat_aug(xf):
        an = jnp.sum(xf * xf, -1)
        return jnp.concatenate(
            [xf, an[:, None], jnp.zeros((N, 7), F32)], axis=1)

    g1 = _gather_rows(xt128, idx_pos.reshape(-1), "conv").reshape(N, NK, 128)
    w1a_2, w1b_2 = _split_w(p["egc2_1"]["W"], 24)
    x1, x1_128, p2 = _comb_egc(p1, g1, xt24, w1b_1, p["egc1_2"]["W"],
                               p["egc1_2"]["b"][None, :],
                               (w1a_2, p["egc2_1"]["b"][None, :]))
    x1a = feat_aug(x1)
    idx1 = _knn33(x1a, x1a.T)
    g2 = _gather_rows(x1_128, idx1.reshape(-1), "conv").reshape(N, NK, 128)
    w1a_3, w1b_3 = _split_w(p["egc3_1"]["W"], 24)
    x2, x2_128, p3 = _comb_egc(p2, g2, x1, w1b_2, p["egc2_2"]["W"],
                               p["egc2_2"]["b"][None, :],
                               (w1a_3, p["egc3_1"]["b"][None, :]))
    x2a = feat_aug(x2)
    idx2 = _knn33(x2a, x2a.T)
    g3 = _gather_rows(x2_128, idx2.reshape(-1), "conv").reshape(N, NK, 128)
    w1a_d1, w1b_d1 = _split_w(p["d1_1"]["W"], 60)
    xloc, xmid, p_d1, xmid128 = _comb3(
        p3, g3, x2, w1b_3, p["egc3_2"]["W"], p["egc3_2"]["b"][None, :], x1,
        p["local_hidden"]["W"], p["local_hidden"]["b"][None, :],
        w1a_d1, p["d1_1"]["b"][None, :])

    idx_all = [idx_d1, idx_d2, idx_d3, idx_d4]
    w1bs = {"d1": w1b_d1}
    for nm in ["d2", "d3", "d4"]:
        _, w1bs[nm] = _split_w(p[nm + "_1"]["W"], 60)
    feat = xmid
    table = xmid128
    p_cur = p_d1
    xds = []
    for i, nm in enumerate(["d1", "d2", "d3", "d4"]):
        gd = _gather_rows(table, idx_all[i].reshape(-1), "conv").reshape(
            N, NK, 128)
        if i < 3:
            nxt = ["d2", "d3", "d4"][i]
            w1a_n, _ = _split_w(p[nxt + "_1"]["W"], 60)
            xd, table, p_cur = _comb_dilated(
                p_cur, gd, feat, w1bs[nm], p[nm + "_2"]["W"],
                p[nm + "_2"]["b"][None, :],
                (w1a_n, p[nxt + "_1"]["b"][None, :]))
        else:
            (xd,) = _comb_dilated(
                p_cur, gd, feat, w1bs[nm], p[nm + "_2"]["W"],
                p[nm + "_2"]["b"][None, :], None)
        xds.append(xd)
        feat = xd

    logits, tgtf = _head_a(xmid, xds[0], xds[1], xds[2], xds[3], p)
    nb_table = jnp.concatenate(
        [pos2d, tgtf[:, 0:1], jnp.zeros((N, 124), F32)], axis=1)
    nb_g = _gather_rows(nb_table, nidx.reshape(-1), "small").reshape(
        N, 12, 128)

    out = _head_c(xloc, xmid, xds, logits, tgtf, nb_g, pos_pad16, p)
    return out[None]
